# Initial kernel scaffold; baseline (speedup 1.0000x reference)
import jax
import jax.numpy as jnp
from jax.experimental import pallas as pl


def kernel(token_ids, weight):
    raise NotImplementedError("write your pallas kernel here")


# speedup vs baseline: 1.7961x; 1.7961x over previous
import functools

import jax
import jax.numpy as jnp
from jax import lax
from jax.experimental import pallas as pl
from jax.experimental.pallas import tpu as pltpu
from jax.experimental.pallas import tpu_sc as plsc

NUM_EMBEDDINGS = 1000000
EMBEDDING_DIM = 64
BATCH = 16384
HIST_LEN = 50

NC = 2
NS = 16
NW = NC * NS

IDX_PER_STREAM = 128
TOTAL = BATCH * HIST_LEN
NROWS = TOTAL // IDX_PER_STREAM
ROWS_PER_W = NROWS // NW
CH = 4
STEPS = ROWS_PER_W // CH
CHUNK = CH * IDX_PER_STREAM


def _body(idx_hbm, w_hbm, out_hbm, idx_v, rows_v, sem):
    wid = lax.axis_index("s") * NC + lax.axis_index("c")
    row0 = wid * ROWS_PER_W

    def step(g, carry):
        r = row0 + g * CH
        pltpu.sync_copy(idx_hbm.at[pl.ds(r, CH)], idx_v)
        copies = []
        for j in range(CH):
            copies.append(pltpu.async_copy(
                w_hbm.at[idx_v.at[j]],
                rows_v.at[pl.ds(j * IDX_PER_STREAM, IDX_PER_STREAM)],
                sem))
        for c in copies:
            c.wait()
        pltpu.sync_copy(rows_v, out_hbm.at[pl.ds(r * IDX_PER_STREAM, CHUNK)])
        return carry

    lax.fori_loop(0, STEPS, step, 0)


@jax.jit
def _gather(idx2d, weight):
    mesh = plsc.VectorSubcoreMesh(core_axis_name="c", subcore_axis_name="s")
    f = pl.kernel(
        _body,
        out_type=jax.ShapeDtypeStruct((TOTAL, EMBEDDING_DIM), jnp.float32),
        mesh=mesh,
        compiler_params=pltpu.CompilerParams(use_tc_tiling_on_sc=False),
        scratch_types=[
            pltpu.VMEM((CH, IDX_PER_STREAM), jnp.int32),
            pltpu.VMEM((CHUNK, EMBEDDING_DIM), jnp.float32),
            pltpu.SemaphoreType.DMA,
        ],
    )
    return f(idx2d, weight)


def kernel(token_ids, weight):
    idx2d = token_ids.astype(jnp.int32).reshape(NROWS, IDX_PER_STREAM)
    out = _gather(idx2d, weight)
    return out.reshape(BATCH, HIST_LEN, EMBEDDING_DIM)
